# baseline (speedup 1.0000x reference)
import functools

import jax
import jax.numpy as jnp
from jax import lax
from jax.experimental import pallas as pl
from jax.experimental.pallas import tpu as pltpu

_C1 = 64
_N2 = 512
_K_LIST = [16, 32, 64]
_R_LIST = [0.2, 0.4, 0.8]
_EPS = 1e-5
_CPAD = 128


def _fps_body(xt_ref, yt_ref, zt_ref, idx_ref, cx_ref, cy_ref, cz_ref):
    B, N = xt_ref.shape
    xt = xt_ref[:]
    yt = yt_ref[:]
    zt = zt_ref[:]
    iota_n = lax.broadcasted_iota(jnp.int32, (B, N), 1)
    iota_s = lax.broadcasted_iota(jnp.int32, (B, _N2), 1)

    def body(i, carry):
        distance, farthest, acc_i, acc_x, acc_y, acc_z = carry
        stepi = (iota_s == i).astype(jnp.int32)
        stepf = stepi.astype(jnp.float32)
        acc_i = acc_i + stepi * farthest
        msk = iota_n == farthest
        cx = jnp.sum(jnp.where(msk, xt, 0.0), axis=1, keepdims=True)
        cy = jnp.sum(jnp.where(msk, yt, 0.0), axis=1, keepdims=True)
        cz = jnp.sum(jnp.where(msk, zt, 0.0), axis=1, keepdims=True)
        acc_x = acc_x + stepf * cx
        acc_y = acc_y + stepf * cy
        acc_z = acc_z + stepf * cz
        dx = xt - cx
        dy = yt - cy
        dz = zt - cz
        dist = (dx * dx + dy * dy) + dz * dz
        distance = jnp.minimum(distance, dist)
        m = jnp.max(distance, axis=1, keepdims=True)
        farthest = jnp.min(
            jnp.where(distance == m, iota_n, N), axis=1, keepdims=True
        ).astype(jnp.int32)
        return distance, farthest, acc_i, acc_x, acc_y, acc_z

    dist0 = xt * 0.0 + 1e10
    far0 = (xt[:, :1] * 0.0).astype(jnp.int32)
    zf = xt[:, :_N2] * 0.0
    zi = zf.astype(jnp.int32)
    _, _, acc_i, acc_x, acc_y, acc_z = lax.fori_loop(
        0, _N2, body, (dist0, far0, zi, zf, zf, zf)
    )
    idx_ref[:] = acc_i
    cx_ref[:] = acc_x
    cy_ref[:] = acc_y
    cz_ref[:] = acc_z


def _fps(xyz):
    B, N, _ = xyz.shape
    xt = jnp.transpose(xyz, (2, 0, 1))
    out = pl.pallas_call(
        _fps_body,
        out_shape=(
            jax.ShapeDtypeStruct((B, _N2), jnp.int32),
            jax.ShapeDtypeStruct((B, _N2), jnp.float32),
            jax.ShapeDtypeStruct((B, _N2), jnp.float32),
            jax.ShapeDtypeStruct((B, _N2), jnp.float32),
        ),
    )(xt[0], xt[1], xt[2])
    fps_idx, cx, cy, cz = out
    new_xyz = jnp.stack([cx, cy, cz], axis=-1)
    return fps_idx, new_xyz


def _sqd_body(nxyz_ref, xyzt_ref, e_ref, lt_ref, out_ref, nc_ref):
    nt = nxyz_ref[0]
    xt = xyzt_ref[0]
    d = jnp.dot(nt, xt, preferred_element_type=jnp.float32)
    d = -2.0 * d
    d = d + jnp.sum(nt * nt, axis=1, keepdims=True)
    d = d + jnp.sum(xt * xt, axis=0, keepdims=True)
    out_ref[0] = d
    e = e_ref[:]
    lt = lt_ref[:]
    nchunk = e.shape[1]
    ncs = []
    for (k, r) in zip(_K_LIST, _R_LIST):
        m = (d <= jnp.float32(r * r)).astype(jnp.bfloat16)
        cnt = jnp.dot(m, e, preferred_element_type=jnp.float32)
        cum = jnp.dot(cnt.astype(jnp.bfloat16), lt,
                      preferred_element_type=jnp.float32)
        ncj = 1.0 + jnp.sum((cum < k).astype(jnp.float32), axis=1,
                            keepdims=True)
        ncs.append(jnp.minimum(ncj, float(nchunk)))
    nc = jnp.concatenate(ncs + [ncs[0]] * 13, axis=1).astype(jnp.int32)
    nc_ref[0] = nc


def _sqdist_all(new_xyz, xyzt):
    B, S, _ = new_xyz.shape
    N = xyzt.shape[2]
    NCHUNK = N // 16
    e = (jnp.arange(N)[:, None] // 16 == jnp.arange(NCHUNK)[None, :]
         ).astype(jnp.bfloat16)
    lt = (jnp.arange(NCHUNK)[:, None] <= jnp.arange(NCHUNK)[None, :]
          ).astype(jnp.bfloat16)
    return pl.pallas_call(
        _sqd_body,
        grid=(B,),
        in_specs=[
            pl.BlockSpec((1, S, 3), lambda b: (b, 0, 0)),
            pl.BlockSpec((1, 3, N), lambda b: (b, 0, 0)),
            pl.BlockSpec((N, NCHUNK), lambda b: (0, 0)),
            pl.BlockSpec((NCHUNK, NCHUNK), lambda b: (0, 0)),
        ],
        out_specs=(
            pl.BlockSpec((1, S, N), lambda b: (b, 0, 0)),
            pl.BlockSpec((1, S, 16), lambda b: (b, 0, 0)),
        ),
        out_shape=(
            jax.ShapeDtypeStruct((B, S, N), jnp.float32),
            jax.ShapeDtypeStruct((B, S, 16), jnp.int32),
        ),
    )(new_xyz, xyzt, e, lt)


def _mlp_body(rows_ref, nxyz_ref, w1_ref, w2_ref, w3_ref, gb_ref, out_ref,
              *, k, cs):
    TR = rows_ref.shape[0]
    g = TR // k
    c1, c2, c3 = cs
    X = rows_ref[:]
    nx = nxyz_ref[:]
    rsq = jnp.sqrt(1.0 + _EPS)
    g1 = gb_ref[0, :c1]
    b1 = gb_ref[1, :c1]
    g2 = gb_ref[2, :c2]
    b2 = gb_ref[3, :c2]
    g3 = gb_ref[4, :c3]
    b3 = gb_ref[5, :c3]

    x = jnp.dot(X.astype(jnp.bfloat16), w1_ref[:],
                preferred_element_type=jnp.float32)
    corr = jnp.dot(nx.astype(jnp.bfloat16), w1_ref[0:3, :],
                   preferred_element_type=jnp.float32)
    x = x.reshape(g, k, c1) - corr[:, None, :]
    x = x.reshape(TR, c1)
    x = jax.nn.relu(g1[None, :] * x / rsq + b1[None, :])
    x = jnp.dot(x.astype(jnp.bfloat16), w2_ref[:],
                preferred_element_type=jnp.float32)
    x = jax.nn.relu(g2[None, :] * x / rsq + b2[None, :])
    x = jnp.dot(x.astype(jnp.bfloat16), w3_ref[:],
                preferred_element_type=jnp.float32)
    x = jax.nn.relu(g3[None, :] * x / rsq + b3[None, :])
    out_ref[:] = jnp.max(x.reshape(g, k, c3), axis=1)


def _mlp_maxpool(rows, nxyz_flat, layer_params, k):
    R = rows.shape[0]
    BS = nxyz_flat.shape[0]
    (W1, g1, b1), (W2, g2, b2), (W3, g3, b3) = layer_params
    c1, c2, c3 = W1.shape[0], W2.shape[0], W3.shape[0]
    w1 = (jnp.zeros((_CPAD, c1), W1.dtype).at[: W1.shape[1], :].set(W1.T)
          ).astype(jnp.bfloat16)
    w2 = W2.T.astype(jnp.bfloat16)
    w3 = W3.T.astype(jnp.bfloat16)
    cmax = max(c1, c2, c3)
    gb = jnp.zeros((6, cmax), jnp.float32)
    for i, v in enumerate((g1, b1, g2, b2, g3, b3)):
        gb = gb.at[i, : v.shape[0]].set(v)
    TR = 2048
    grid = (R // TR,)
    gpt = TR // k
    return pl.pallas_call(
        functools.partial(_mlp_body, k=k, cs=(c1, c2, c3)),
        grid=grid,
        in_specs=[
            pl.BlockSpec((TR, _CPAD), lambda i: (i, 0)),
            pl.BlockSpec((gpt, 3), lambda i: (i, 0)),
            pl.BlockSpec(w1.shape, lambda i: (0, 0)),
            pl.BlockSpec(w2.shape, lambda i: (0, 0)),
            pl.BlockSpec(w3.shape, lambda i: (0, 0)),
            pl.BlockSpec(gb.shape, lambda i: (0, 0)),
        ],
        out_specs=pl.BlockSpec((gpt, c3), lambda i: (i, 0)),
        out_shape=jax.ShapeDtypeStruct((BS, c3), jnp.float32),
    )(rows, nxyz_flat, w1, w2, w3, gb)


def _sc_group_gather(sqd, ncs, table, B, N, S):
    from jax.experimental.pallas import tpu_sc as plsc

    NW = 32
    RPW = (B * S) // NW
    CH = 128
    L = 16
    scales = [(k, float(r * r)) for k, r in zip(_K_LIST, _R_LIST)]

    mesh = plsc.VectorSubcoreMesh(core_axis_name="c", subcore_axis_name="s")

    import struct

    r2bits = [struct.unpack("<i", struct.pack("<f", r2))[0]
              for (_, r2) in scales]

    def body(sqd_ref, nc_ref, table_ref, out0_ref, out1_ref, out2_ref,
             dbuf, ncbuf, idxbuf, gacc0, gacc1, gacc2, rowbuf, semg):
        wid = lax.axis_index("s") * 2 + lax.axis_index("c")
        nbase = ((wid * RPW) // S) * N
        gaccs = [gacc0, gacc1, gacc2]
        outs = [out0_ref, out1_ref, out2_ref]
        ids0 = lax.iota(jnp.int32, L)

        def row_body(i, _):
            r = wid * RPW + i
            pltpu.sync_copy(sqd_ref.at[r], dbuf)
            pltpu.sync_copy(nc_ref.at[r], ncbuf)
            ncv = ncbuf[pl.ds(0, L)]
            for j, (k, _) in enumerate(scales):
                gacc = gaccs[j]
                r2i = r2bits[j]
                nc_j = ncv[j]

                def hex_body(p, off, r2i=r2i):
                    e = p * 64
                    base = pl.multiple_of(e, 64)
                    vs = [dbuf[pl.ds(base + u, 1)] for u in range(64)]
                    for u in range(64):
                        idxbuf[pl.ds(off, 1)] = lax.broadcast(e + u, (1,))
                        off = off + (vs[u][0] <= r2i).astype(jnp.int32)
                    return off

                off = lax.fori_loop(0, (nc_j + 3) // 4, hex_body,
                                    jnp.int32(0))
                fv = idxbuf[pl.ds(0, 1)][0]
                for q in range(k // L):
                    chunk = idxbuf[pl.ds(q * L, L)]
                    idsq = ids0 + q * L
                    sel = jnp.where(idsq < off, chunk, fv) + nbase
                    gacc[pl.ds(pl.multiple_of(i * k + q * L, L), L)] = sel
            return 0

        lax.fori_loop(0, RPW, row_body, 0)

        for j, (k, _) in enumerate(scales):
            gacc = gaccs[j]
            out = outs[j]

            def chunk_body(c, _, gacc=gacc, out=out, k=k):
                src = pl.multiple_of(c * CH, CH)
                dst = pl.multiple_of(wid * (RPW * k) + c * CH, CH)
                pltpu.async_copy(
                    table_ref.at[gacc.at[pl.ds(src, CH)]], rowbuf, semg
                ).wait()
                pltpu.sync_copy(rowbuf, out.at[pl.ds(dst, CH)])
                return 0

            lax.fori_loop(0, (RPW * k) // CH, chunk_body, 0)

    k0, k1, k2 = _K_LIST
    fn = pl.kernel(
        body,
        out_type=(
            jax.ShapeDtypeStruct((B * S * k0, _CPAD), jnp.float32),
            jax.ShapeDtypeStruct((B * S * k1, _CPAD), jnp.float32),
            jax.ShapeDtypeStruct((B * S * k2, _CPAD), jnp.float32),
        ),
        mesh=mesh,
        scratch_types=[
            pltpu.VMEM((N,), jnp.int32),
            pltpu.VMEM((16,), jnp.int32),
            pltpu.VMEM((192,), jnp.int32),
            pltpu.VMEM((RPW * k0,), jnp.int32),
            pltpu.VMEM((RPW * k1,), jnp.int32),
            pltpu.VMEM((RPW * k2,), jnp.int32),
            pltpu.VMEM((CH, _CPAD), jnp.float32),
            pltpu.SemaphoreType.DMA,
        ],
    )
    return fn(sqd, ncs, table)


def kernel(xyz, feature, params):
    B, N, _ = xyz.shape
    S = _N2
    xyzt = jnp.transpose(xyz, (0, 2, 1))
    _, new_xyz = _fps(xyz)
    nxyz_flat = new_xyz.reshape(B * S, 3)
    sqrd, ncs = _sqdist_all(new_xyz, xyzt)

    table = jnp.concatenate(
        [xyz, feature, jnp.zeros((B, N, _CPAD - 3 - _C1), jnp.float32)], axis=-1
    ).reshape(B * N, _CPAD)

    rows_all = _sc_group_gather(
        sqrd.reshape(B * S, N).view(jnp.int32), ncs.reshape(B * S, 16),
        table, B, N, S
    )
    feats = []
    for (k, layer_params, rows) in zip(_K_LIST, params, rows_all):
        out = _mlp_maxpool(rows, nxyz_flat, layer_params, k)
        feats.append(out.reshape(B, S, -1))
    return jnp.concatenate(feats, axis=-1)


# speedup vs baseline: 1.0548x; 1.0548x over previous
import functools

import jax
import jax.numpy as jnp
from jax import lax
from jax.experimental import pallas as pl
from jax.experimental.pallas import tpu as pltpu

_C1 = 64
_N2 = 512
_K_LIST = [16, 32, 64]
_R_LIST = [0.2, 0.4, 0.8]
_EPS = 1e-5
_CPAD = 128


def _fps_body(xt_ref, yt_ref, zt_ref, idx_ref, cx_ref, cy_ref, cz_ref):
    B, N = xt_ref.shape
    xt = xt_ref[:]
    yt = yt_ref[:]
    zt = zt_ref[:]
    iota_n = lax.broadcasted_iota(jnp.int32, (B, N), 1)
    iota_s = lax.broadcasted_iota(jnp.int32, (B, _N2), 1)

    def body(i, carry):
        distance, farthest, acc_i, acc_x, acc_y, acc_z = carry
        stepi = (iota_s == i).astype(jnp.int32)
        stepf = stepi.astype(jnp.float32)
        acc_i = acc_i + stepi * farthest
        msk = iota_n == farthest
        cx = jnp.sum(jnp.where(msk, xt, 0.0), axis=1, keepdims=True)
        cy = jnp.sum(jnp.where(msk, yt, 0.0), axis=1, keepdims=True)
        cz = jnp.sum(jnp.where(msk, zt, 0.0), axis=1, keepdims=True)
        acc_x = acc_x + stepf * cx
        acc_y = acc_y + stepf * cy
        acc_z = acc_z + stepf * cz
        dx = xt - cx
        dy = yt - cy
        dz = zt - cz
        dist = (dx * dx + dy * dy) + dz * dz
        distance = jnp.minimum(distance, dist)
        m = jnp.max(distance, axis=1, keepdims=True)
        farthest = jnp.min(
            jnp.where(distance == m, iota_n, N), axis=1, keepdims=True
        ).astype(jnp.int32)
        return distance, farthest, acc_i, acc_x, acc_y, acc_z

    dist0 = xt * 0.0 + 1e10
    far0 = (xt[:, :1] * 0.0).astype(jnp.int32)
    zf = xt[:, :_N2] * 0.0
    zi = zf.astype(jnp.int32)
    _, _, acc_i, acc_x, acc_y, acc_z = lax.fori_loop(
        0, _N2, body, (dist0, far0, zi, zf, zf, zf)
    )
    idx_ref[:] = acc_i
    cx_ref[:] = acc_x
    cy_ref[:] = acc_y
    cz_ref[:] = acc_z


def _fps(xyz):
    B, N, _ = xyz.shape
    xt = jnp.transpose(xyz, (2, 0, 1))
    out = pl.pallas_call(
        _fps_body,
        out_shape=(
            jax.ShapeDtypeStruct((B, _N2), jnp.int32),
            jax.ShapeDtypeStruct((B, _N2), jnp.float32),
            jax.ShapeDtypeStruct((B, _N2), jnp.float32),
            jax.ShapeDtypeStruct((B, _N2), jnp.float32),
        ),
    )(xt[0], xt[1], xt[2])
    fps_idx, cx, cy, cz = out
    new_xyz = jnp.stack([cx, cy, cz], axis=-1)
    return fps_idx, new_xyz


def _sqd_body(nxyz_ref, xyzt_ref, e_ref, lt_ref, out_ref, nc_ref):
    nt = nxyz_ref[0]
    xt = xyzt_ref[0]
    d = jnp.dot(nt, xt, preferred_element_type=jnp.float32)
    d = -2.0 * d
    d = d + jnp.sum(nt * nt, axis=1, keepdims=True)
    d = d + jnp.sum(xt * xt, axis=0, keepdims=True)
    out_ref[0] = d
    e = e_ref[:]
    lt = lt_ref[:]
    nchunk = e.shape[1]
    ncs = []
    for (k, r) in zip(_K_LIST, _R_LIST):
        m = (d <= jnp.float32(r * r)).astype(jnp.bfloat16)
        cnt = jnp.dot(m, e, preferred_element_type=jnp.float32)
        cum = jnp.dot(cnt.astype(jnp.bfloat16), lt,
                      preferred_element_type=jnp.float32)
        ncj = 1.0 + jnp.sum((cum < k).astype(jnp.float32), axis=1,
                            keepdims=True)
        ncs.append(jnp.minimum(ncj, float(nchunk)))
    nc = jnp.concatenate(ncs + [ncs[0]] * 13, axis=1).astype(jnp.int32)
    nc_ref[0] = nc


def _sqdist_all(new_xyz, xyzt):
    B, S, _ = new_xyz.shape
    N = xyzt.shape[2]
    NCHUNK = N // 16
    e = (jnp.arange(N)[:, None] // 16 == jnp.arange(NCHUNK)[None, :]
         ).astype(jnp.bfloat16)
    lt = (jnp.arange(NCHUNK)[:, None] <= jnp.arange(NCHUNK)[None, :]
          ).astype(jnp.bfloat16)
    return pl.pallas_call(
        _sqd_body,
        grid=(B,),
        in_specs=[
            pl.BlockSpec((1, S, 3), lambda b: (b, 0, 0)),
            pl.BlockSpec((1, 3, N), lambda b: (b, 0, 0)),
            pl.BlockSpec((N, NCHUNK), lambda b: (0, 0)),
            pl.BlockSpec((NCHUNK, NCHUNK), lambda b: (0, 0)),
        ],
        out_specs=(
            pl.BlockSpec((1, S, N), lambda b: (b, 0, 0)),
            pl.BlockSpec((1, S, 16), lambda b: (b, 0, 0)),
        ),
        out_shape=(
            jax.ShapeDtypeStruct((B, S, N), jnp.float32),
            jax.ShapeDtypeStruct((B, S, 16), jnp.int32),
        ),
    )(new_xyz, xyzt, e, lt)


def _mlp_body(rows_ref, nxyz_ref, w1_ref, w2_ref, w3_ref, gb_ref, out_ref,
              *, k, cs):
    TR = rows_ref.shape[0]
    g = TR // k
    c1, c2, c3 = cs
    X = rows_ref[:]
    nx = nxyz_ref[:]
    rsq = jnp.sqrt(1.0 + _EPS)
    g1 = gb_ref[0, :c1]
    b1 = gb_ref[1, :c1]
    g2 = gb_ref[2, :c2]
    b2 = gb_ref[3, :c2]
    g3 = gb_ref[4, :c3]
    b3 = gb_ref[5, :c3]

    x = jnp.dot(X.astype(jnp.bfloat16), w1_ref[:],
                preferred_element_type=jnp.float32)
    corr = jnp.dot(nx.astype(jnp.bfloat16), w1_ref[0:3, :],
                   preferred_element_type=jnp.float32)
    x = x.reshape(g, k, c1) - corr[:, None, :]
    x = x.reshape(TR, c1)
    x = jax.nn.relu(g1[None, :] * x / rsq + b1[None, :])
    x = jnp.dot(x.astype(jnp.bfloat16), w2_ref[:],
                preferred_element_type=jnp.float32)
    x = jax.nn.relu(g2[None, :] * x / rsq + b2[None, :])
    x = jnp.dot(x.astype(jnp.bfloat16), w3_ref[:],
                preferred_element_type=jnp.float32)
    x = jax.nn.relu(g3[None, :] * x / rsq + b3[None, :])
    out_ref[:] = jnp.max(x.reshape(g, k, c3), axis=1)


def _mlp_maxpool(rows, nxyz_flat, layer_params, k):
    R = rows.shape[0]
    BS = nxyz_flat.shape[0]
    (W1, g1, b1), (W2, g2, b2), (W3, g3, b3) = layer_params
    c1, c2, c3 = W1.shape[0], W2.shape[0], W3.shape[0]
    w1 = (jnp.zeros((_CPAD, c1), W1.dtype).at[: W1.shape[1], :].set(W1.T)
          ).astype(jnp.bfloat16)
    w2 = W2.T.astype(jnp.bfloat16)
    w3 = W3.T.astype(jnp.bfloat16)
    cmax = max(c1, c2, c3)
    gb = jnp.zeros((6, cmax), jnp.float32)
    for i, v in enumerate((g1, b1, g2, b2, g3, b3)):
        gb = gb.at[i, : v.shape[0]].set(v)
    TR = 2048
    grid = (R // TR,)
    gpt = TR // k
    return pl.pallas_call(
        functools.partial(_mlp_body, k=k, cs=(c1, c2, c3)),
        grid=grid,
        in_specs=[
            pl.BlockSpec((TR, _CPAD), lambda i: (i, 0)),
            pl.BlockSpec((gpt, 3), lambda i: (i, 0)),
            pl.BlockSpec(w1.shape, lambda i: (0, 0)),
            pl.BlockSpec(w2.shape, lambda i: (0, 0)),
            pl.BlockSpec(w3.shape, lambda i: (0, 0)),
            pl.BlockSpec(gb.shape, lambda i: (0, 0)),
        ],
        out_specs=pl.BlockSpec((gpt, c3), lambda i: (i, 0)),
        out_shape=jax.ShapeDtypeStruct((BS, c3), jnp.float32),
    )(rows, nxyz_flat, w1, w2, w3, gb)


def _sc_group_gather(sqd, ncs, table, B, N, S):
    from jax.experimental.pallas import tpu_sc as plsc

    NW = 32
    RPW = (B * S) // NW
    CH = 128
    L = 16
    scales = [(k, float(r * r)) for k, r in zip(_K_LIST, _R_LIST)]

    mesh = plsc.VectorSubcoreMesh(core_axis_name="c", subcore_axis_name="s")

    import struct

    r2bits = [struct.unpack("<i", struct.pack("<f", r2))[0]
              for (_, r2) in scales]

    def body(sqd_ref, nc_ref, table_ref, out0_ref, out1_ref, out2_ref,
             dbuf, ncbuf, idxbuf, gacc0, gacc1, gacc2, rowbuf, semg):
        wid = lax.axis_index("s") * 2 + lax.axis_index("c")
        nbase = ((wid * RPW) // S) * N
        gaccs = [gacc0, gacc1, gacc2]
        outs = [out0_ref, out1_ref, out2_ref]
        ids0 = lax.iota(jnp.int32, L)

        def row_body(i, _):
            r = wid * RPW + i
            pltpu.sync_copy(sqd_ref.at[r], dbuf)
            pltpu.sync_copy(nc_ref.at[r], ncbuf)
            ncv = ncbuf[pl.ds(0, L)]
            for j, (k, _) in enumerate(scales):
                gacc = gaccs[j]
                r2i = r2bits[j]
                nc_j = ncv[j]

                def hex_body(p, off, r2i=r2i):
                    e = p * 32
                    base = pl.multiple_of(e, 32)
                    vs = [dbuf[pl.ds(base + u, 1)] for u in range(32)]
                    for u in range(32):
                        idxbuf[pl.ds(off, 1)] = lax.broadcast(e + u, (1,))
                        off = off + (vs[u][0] <= r2i).astype(jnp.int32)
                    return off

                off = lax.fori_loop(0, (nc_j + 1) // 2, hex_body,
                                    jnp.int32(0))
                fv = idxbuf[pl.ds(0, 1)][0]
                for q in range(k // L):
                    chunk = idxbuf[pl.ds(q * L, L)]
                    idsq = ids0 + q * L
                    sel = jnp.where(idsq < off, chunk, fv) + nbase
                    gacc[pl.ds(pl.multiple_of(i * k + q * L, L), L)] = sel
            return 0

        lax.fori_loop(0, RPW, row_body, 0)

        for j, (k, _) in enumerate(scales):
            gacc = gaccs[j]
            out = outs[j]

            def chunk_body(c, _, gacc=gacc, out=out, k=k):
                src = pl.multiple_of(c * CH, CH)
                dst = pl.multiple_of(wid * (RPW * k) + c * CH, CH)
                pltpu.async_copy(
                    table_ref.at[gacc.at[pl.ds(src, CH)]], rowbuf, semg
                ).wait()
                pltpu.sync_copy(rowbuf, out.at[pl.ds(dst, CH)])
                return 0

            lax.fori_loop(0, (RPW * k) // CH, chunk_body, 0)

    k0, k1, k2 = _K_LIST
    fn = pl.kernel(
        body,
        out_type=(
            jax.ShapeDtypeStruct((B * S * k0, _CPAD), jnp.float32),
            jax.ShapeDtypeStruct((B * S * k1, _CPAD), jnp.float32),
            jax.ShapeDtypeStruct((B * S * k2, _CPAD), jnp.float32),
        ),
        mesh=mesh,
        scratch_types=[
            pltpu.VMEM((N,), jnp.int32),
            pltpu.VMEM((16,), jnp.int32),
            pltpu.VMEM((192,), jnp.int32),
            pltpu.VMEM((RPW * k0,), jnp.int32),
            pltpu.VMEM((RPW * k1,), jnp.int32),
            pltpu.VMEM((RPW * k2,), jnp.int32),
            pltpu.VMEM((CH, _CPAD), jnp.float32),
            pltpu.SemaphoreType.DMA,
        ],
    )
    return fn(sqd, ncs, table)


def kernel(xyz, feature, params):
    B, N, _ = xyz.shape
    S = _N2
    xyzt = jnp.transpose(xyz, (0, 2, 1))
    _, new_xyz = _fps(xyz)
    nxyz_flat = new_xyz.reshape(B * S, 3)
    sqrd, ncs = _sqdist_all(new_xyz, xyzt)

    table = jnp.concatenate(
        [xyz, feature, jnp.zeros((B, N, _CPAD - 3 - _C1), jnp.float32)], axis=-1
    ).reshape(B * N, _CPAD)

    rows_all = _sc_group_gather(
        sqrd.reshape(B * S, N).view(jnp.int32), ncs.reshape(B * S, 16),
        table, B, N, S
    )
    feats = []
    for (k, layer_params, rows) in zip(_K_LIST, params, rows_all):
        out = _mlp_maxpool(rows, nxyz_flat, layer_params, k)
        feats.append(out.reshape(B, S, -1))
    return jnp.concatenate(feats, axis=-1)


# speedup vs baseline: 1.1903x; 1.1285x over previous
import functools

import jax
import jax.numpy as jnp
from jax import lax
from jax.experimental import pallas as pl
from jax.experimental.pallas import tpu as pltpu

_C1 = 64
_N2 = 512
_K_LIST = [16, 32, 64]
_R_LIST = [0.2, 0.4, 0.8]
_EPS = 1e-5
_CPAD = 128


def _fps_body(xt_ref, yt_ref, zt_ref, idx_ref, cx_ref, cy_ref, cz_ref):
    B, N = xt_ref.shape
    xt = xt_ref[:]
    yt = yt_ref[:]
    zt = zt_ref[:]
    iota_n = lax.broadcasted_iota(jnp.int32, (B, N), 1)
    iota_s = lax.broadcasted_iota(jnp.int32, (B, _N2), 1)

    def body(i, carry):
        distance, farthest, acc_i, acc_x, acc_y, acc_z = carry
        stepi = (iota_s == i).astype(jnp.int32)
        stepf = stepi.astype(jnp.float32)
        acc_i = acc_i + stepi * farthest
        msk = iota_n == farthest
        cx = jnp.sum(jnp.where(msk, xt, 0.0), axis=1, keepdims=True)
        cy = jnp.sum(jnp.where(msk, yt, 0.0), axis=1, keepdims=True)
        cz = jnp.sum(jnp.where(msk, zt, 0.0), axis=1, keepdims=True)
        acc_x = acc_x + stepf * cx
        acc_y = acc_y + stepf * cy
        acc_z = acc_z + stepf * cz
        dx = xt - cx
        dy = yt - cy
        dz = zt - cz
        dist = (dx * dx + dy * dy) + dz * dz
        distance = jnp.minimum(distance, dist)
        m = jnp.max(distance, axis=1, keepdims=True)
        farthest = jnp.min(
            jnp.where(distance == m, iota_n, N), axis=1, keepdims=True
        ).astype(jnp.int32)
        return distance, farthest, acc_i, acc_x, acc_y, acc_z

    dist0 = xt * 0.0 + 1e10
    far0 = (xt[:, :1] * 0.0).astype(jnp.int32)
    zf = xt[:, :_N2] * 0.0
    zi = zf.astype(jnp.int32)
    _, _, acc_i, acc_x, acc_y, acc_z = lax.fori_loop(
        0, _N2, body, (dist0, far0, zi, zf, zf, zf)
    )
    idx_ref[:] = acc_i
    cx_ref[:] = acc_x
    cy_ref[:] = acc_y
    cz_ref[:] = acc_z


def _fps(xyz):
    B, N, _ = xyz.shape
    xt = jnp.transpose(xyz, (2, 0, 1))
    out = pl.pallas_call(
        _fps_body,
        out_shape=(
            jax.ShapeDtypeStruct((B, _N2), jnp.int32),
            jax.ShapeDtypeStruct((B, _N2), jnp.float32),
            jax.ShapeDtypeStruct((B, _N2), jnp.float32),
            jax.ShapeDtypeStruct((B, _N2), jnp.float32),
        ),
    )(xt[0], xt[1], xt[2])
    fps_idx, cx, cy, cz = out
    new_xyz = jnp.stack([cx, cy, cz], axis=-1)
    return fps_idx, new_xyz


def _sqd_body(nxyz_ref, xyzt_ref, e_ref, lt_ref, out_ref, nc_ref):
    nt = nxyz_ref[0]
    xt = xyzt_ref[0]
    d = jnp.dot(nt, xt, preferred_element_type=jnp.float32)
    d = -2.0 * d
    d = d + jnp.sum(nt * nt, axis=1, keepdims=True)
    d = d + jnp.sum(xt * xt, axis=0, keepdims=True)
    out_ref[0] = d
    e = e_ref[:]
    lt = lt_ref[:]
    nchunk = e.shape[1]
    ncs = []
    for (k, r) in zip(_K_LIST, _R_LIST):
        m = (d <= jnp.float32(r * r)).astype(jnp.bfloat16)
        cnt = jnp.dot(m, e, preferred_element_type=jnp.float32)
        cum = jnp.dot(cnt.astype(jnp.bfloat16), lt,
                      preferred_element_type=jnp.float32)
        ncj = 1.0 + jnp.sum((cum < k).astype(jnp.float32), axis=1,
                            keepdims=True)
        ncs.append(jnp.minimum(ncj, float(nchunk)))
    nc = jnp.concatenate(ncs + [ncs[0]] * 13, axis=1).astype(jnp.int32)
    nc_ref[0] = nc


def _sqdist_all(new_xyz, xyzt):
    B, S, _ = new_xyz.shape
    N = xyzt.shape[2]
    NCHUNK = N // 16
    e = (jnp.arange(N)[:, None] // 16 == jnp.arange(NCHUNK)[None, :]
         ).astype(jnp.bfloat16)
    lt = (jnp.arange(NCHUNK)[:, None] <= jnp.arange(NCHUNK)[None, :]
          ).astype(jnp.bfloat16)
    return pl.pallas_call(
        _sqd_body,
        grid=(B,),
        in_specs=[
            pl.BlockSpec((1, S, 3), lambda b: (b, 0, 0)),
            pl.BlockSpec((1, 3, N), lambda b: (b, 0, 0)),
            pl.BlockSpec((N, NCHUNK), lambda b: (0, 0)),
            pl.BlockSpec((NCHUNK, NCHUNK), lambda b: (0, 0)),
        ],
        out_specs=(
            pl.BlockSpec((1, S, N), lambda b: (b, 0, 0)),
            pl.BlockSpec((1, S, 16), lambda b: (b, 0, 0)),
        ),
        out_shape=(
            jax.ShapeDtypeStruct((B, S, N), jnp.float32),
            jax.ShapeDtypeStruct((B, S, 16), jnp.int32),
        ),
    )(new_xyz, xyzt, e, lt)


def _mlp_body(rows_ref, nxyz_ref, w1_ref, w2_ref, w3_ref, gb_ref, out_ref,
              *, k, cs):
    TR = rows_ref.shape[0]
    g = TR // k
    c1, c2, c3 = cs
    X = rows_ref[:]
    nx = nxyz_ref[:]
    rsq = jnp.sqrt(1.0 + _EPS)
    g1 = gb_ref[0, :c1]
    b1 = gb_ref[1, :c1]
    g2 = gb_ref[2, :c2]
    b2 = gb_ref[3, :c2]
    g3 = gb_ref[4, :c3]
    b3 = gb_ref[5, :c3]

    x = jnp.dot(X.astype(jnp.bfloat16), w1_ref[:],
                preferred_element_type=jnp.float32)
    corr = jnp.dot(nx.astype(jnp.bfloat16), w1_ref[0:3, :],
                   preferred_element_type=jnp.float32)
    x = x.reshape(g, k, c1) - corr[:, None, :]
    x = x.reshape(TR, c1)
    x = jax.nn.relu(g1[None, :] * x / rsq + b1[None, :])
    x = jnp.dot(x.astype(jnp.bfloat16), w2_ref[:],
                preferred_element_type=jnp.float32)
    x = jax.nn.relu(g2[None, :] * x / rsq + b2[None, :])
    x = jnp.dot(x.astype(jnp.bfloat16), w3_ref[:],
                preferred_element_type=jnp.float32)
    x = jax.nn.relu(g3[None, :] * x / rsq + b3[None, :])
    out_ref[:] = jnp.max(x.reshape(g, k, c3), axis=1)


def _mlp_maxpool(rows, nxyz_flat, layer_params, k):
    R = rows.shape[0]
    BS = nxyz_flat.shape[0]
    (W1, g1, b1), (W2, g2, b2), (W3, g3, b3) = layer_params
    c1, c2, c3 = W1.shape[0], W2.shape[0], W3.shape[0]
    w1 = (jnp.zeros((_CPAD, c1), W1.dtype).at[: W1.shape[1], :].set(W1.T)
          ).astype(jnp.bfloat16)
    w2 = W2.T.astype(jnp.bfloat16)
    w3 = W3.T.astype(jnp.bfloat16)
    cmax = max(c1, c2, c3)
    gb = jnp.zeros((6, cmax), jnp.float32)
    for i, v in enumerate((g1, b1, g2, b2, g3, b3)):
        gb = gb.at[i, : v.shape[0]].set(v)
    TR = 2048
    grid = (R // TR,)
    gpt = TR // k
    return pl.pallas_call(
        functools.partial(_mlp_body, k=k, cs=(c1, c2, c3)),
        grid=grid,
        in_specs=[
            pl.BlockSpec((TR, _CPAD), lambda i: (i, 0)),
            pl.BlockSpec((gpt, 3), lambda i: (i, 0)),
            pl.BlockSpec(w1.shape, lambda i: (0, 0)),
            pl.BlockSpec(w2.shape, lambda i: (0, 0)),
            pl.BlockSpec(w3.shape, lambda i: (0, 0)),
            pl.BlockSpec(gb.shape, lambda i: (0, 0)),
        ],
        out_specs=pl.BlockSpec((gpt, c3), lambda i: (i, 0)),
        out_shape=jax.ShapeDtypeStruct((BS, c3), jnp.float32),
    )(rows, nxyz_flat, w1, w2, w3, gb)


def _sc_group_gather(sqd, ncs, table, B, N, S):
    from jax.experimental.pallas import tpu_sc as plsc

    NW = 32
    RPW = (B * S) // NW
    CH = 128
    L = 16
    scales = [(k, float(r * r)) for k, r in zip(_K_LIST, _R_LIST)]

    mesh = plsc.VectorSubcoreMesh(core_axis_name="c", subcore_axis_name="s")

    import struct

    r2bits = [struct.unpack("<i", struct.pack("<f", r2))[0]
              for (_, r2) in scales]

    def body(sqd_ref, nc_ref, table_ref, out0_ref, out1_ref, out2_ref,
             dbuf, dbuf1, ncbuf, idxbuf, gacc0, gacc1, gacc2, rowbuf,
             semg, semd0, semd1):
        wid = lax.axis_index("s") * 2 + lax.axis_index("c")
        nbase = ((wid * RPW) // S) * N
        gaccs = [gacc0, gacc1, gacc2]
        outs = [out0_ref, out1_ref, out2_ref]
        ids0 = lax.iota(jnp.int32, L)
        rbase = wid * RPW
        pltpu.sync_copy(
            nc_ref.at[pl.ds(pl.multiple_of(rbase * 16, RPW * 16), RPW * 16)],
            ncbuf)
        pltpu.async_copy(sqd_ref.at[rbase], dbuf, semd0)

        def scan_row(i, db, _unused):
            r = rbase + i
            ncv = ncbuf[pl.ds(pl.multiple_of(i * 16, 16), L)]
            for j, (k, _) in enumerate(scales):
                gacc = gaccs[j]
                r2i = r2bits[j]
                nc_j = ncv[j]

                def hex_body(p, off, r2i=r2i, db=db):
                    e = p * 32
                    base = pl.multiple_of(e, 32)
                    vs = [db[pl.ds(base + u, 1)] for u in range(32)]
                    for u in range(32):
                        idxbuf[pl.ds(off, 1)] = lax.broadcast(e + u, (1,))
                        off = off + (vs[u][0] <= r2i).astype(jnp.int32)
                    return off

                off = lax.fori_loop(0, (nc_j + 1) // 2, hex_body,
                                    jnp.int32(0))
                fv = idxbuf[pl.ds(0, 1)][0]
                for q in range(k // L):
                    chunk = idxbuf[pl.ds(q * L, L)]
                    idsq = ids0 + q * L
                    sel = jnp.where(idsq < off, chunk, fv) + nbase
                    gacc[pl.ds(pl.multiple_of(i * k + q * L, L), L)] = sel
            return 0

        def row_pair(ip, _):
            i0 = ip * 2
            r0 = rbase + i0
            pltpu.async_copy(sqd_ref.at[r0 + 1], dbuf1, semd1)
            pltpu.make_async_copy(sqd_ref.at[r0], dbuf, semd0).wait()
            scan_row(i0, dbuf, 0)
            rn = jnp.minimum(r0 + 2, B * S - 1)
            pltpu.async_copy(sqd_ref.at[rn], dbuf, semd0)
            pltpu.make_async_copy(sqd_ref.at[r0 + 1], dbuf1, semd1).wait()
            scan_row(i0 + 1, dbuf1, 0)
            return 0

        lax.fori_loop(0, RPW // 2, row_pair, 0)
        pltpu.make_async_copy(sqd_ref.at[rbase], dbuf, semd0).wait()

        for j, (k, _) in enumerate(scales):
            gacc = gaccs[j]
            out = outs[j]

            def chunk_body(c, _, gacc=gacc, out=out, k=k):
                src = pl.multiple_of(c * CH, CH)
                dst = pl.multiple_of(wid * (RPW * k) + c * CH, CH)
                pltpu.async_copy(
                    table_ref.at[gacc.at[pl.ds(src, CH)]], rowbuf, semg
                ).wait()
                pltpu.sync_copy(rowbuf, out.at[pl.ds(dst, CH)])
                return 0

            lax.fori_loop(0, (RPW * k) // CH, chunk_body, 0)

    k0, k1, k2 = _K_LIST
    fn = pl.kernel(
        body,
        out_type=(
            jax.ShapeDtypeStruct((B * S * k0, _CPAD), jnp.float32),
            jax.ShapeDtypeStruct((B * S * k1, _CPAD), jnp.float32),
            jax.ShapeDtypeStruct((B * S * k2, _CPAD), jnp.float32),
        ),
        mesh=mesh,
        scratch_types=[
            pltpu.VMEM((N,), jnp.int32),
            pltpu.VMEM((N,), jnp.int32),
            pltpu.VMEM((RPW * 16,), jnp.int32),
            pltpu.VMEM((192,), jnp.int32),
            pltpu.VMEM((RPW * k0,), jnp.int32),
            pltpu.VMEM((RPW * k1,), jnp.int32),
            pltpu.VMEM((RPW * k2,), jnp.int32),
            pltpu.VMEM((CH, _CPAD), jnp.float32),
            pltpu.SemaphoreType.DMA,
            pltpu.SemaphoreType.DMA,
            pltpu.SemaphoreType.DMA,
        ],
    )
    return fn(sqd, ncs, table)


def kernel(xyz, feature, params):
    B, N, _ = xyz.shape
    S = _N2
    xyzt = jnp.transpose(xyz, (0, 2, 1))
    _, new_xyz = _fps(xyz)
    nxyz_flat = new_xyz.reshape(B * S, 3)
    sqrd, ncs = _sqdist_all(new_xyz, xyzt)

    table = jnp.concatenate(
        [xyz, feature, jnp.zeros((B, N, _CPAD - 3 - _C1), jnp.float32)], axis=-1
    ).reshape(B * N, _CPAD)

    rows_all = _sc_group_gather(
        sqrd.reshape(B * S, N).view(jnp.int32), ncs.reshape(B * S * 16),
        table, B, N, S
    )
    feats = []
    for (k, layer_params, rows) in zip(_K_LIST, params, rows_all):
        out = _mlp_maxpool(rows, nxyz_flat, layer_params, k)
        feats.append(out.reshape(B, S, -1))
    return jnp.concatenate(feats, axis=-1)
